# baseline (speedup 1.0000x reference)
import functools

import jax
import jax.numpy as jnp
from jax import lax
from jax.experimental import pallas as pl
from jax.experimental.pallas import tpu as pltpu
from jax.experimental.pallas import tpu_sc as plsc

_F32 = jnp.float32
_HI = lax.Precision.DEFAULT


def _taps(K):
    return [(ky, kx) for ky in range(K) for kx in range(K)]


def _conv_plane(x, w_ref, b_ref, K):
    H, W = x.shape
    p = K // 2
    xp = jnp.pad(x, ((p, p), (p, p)))
    pats = jnp.stack([xp[ky:ky + H, kx:kx + W] for ky, kx in _taps(K)],
                     axis=0).reshape(K * K, H * W)
    return jnp.dot(w_ref[...], pats, preferred_element_type=_F32, precision=_HI) + b_ref[...]


def _conv_cmaj(x, w_ref, b_ref, K, res):
    C, H, W = x.shape
    p = K // 2
    Wp = W + 2 * p
    xf = jnp.pad(x, ((0, 0), (p, p + 1), (p, p))).reshape(C, (H + 2 * p + 1) * Wp)
    NN = H * Wp
    pats = jnp.concatenate(
        [xf[:, ky * Wp + kx: ky * Wp + kx + NN] for ky, kx in _taps(K)], axis=0)
    y = jnp.dot(w_ref[...], pats, preferred_element_type=_F32, precision=_HI) + b_ref[...]
    y = jax.nn.relu(y.reshape(y.shape[0], H, Wp)[:, :, :W])
    return x + y if res else y


def _pool(x, s_ref):
    C, H, W = x.shape
    mh = jnp.maximum(x[:, :, :W - 1], x[:, :, 1:])
    r = lax.broadcasted_iota(jnp.int32, (W - 1, W // 2), 0)
    c = lax.broadcasted_iota(jnp.int32, (W - 1, W // 2), 1)
    S = (r == 2 * c).astype(_F32)
    s_ref[...] = jnp.dot(mh.reshape(C * H, W - 1), S,
                         preferred_element_type=_F32,
                         precision=lax.Precision.HIGHEST).reshape(C, H, W // 2)
    return jnp.maximum(s_ref[:, ::2, :], s_ref[:, 1::2, :])


def _enc_body(X_ref, w1_ref, b1_ref, w2_ref, b2_ref, w3_ref, b3_ref,
              w4_ref, b4_ref, w5_ref, b5_ref, w6_ref, b6_ref,
              emb_ref, embT_ref, enc_ref, idx_ref, p1, p2, p3):
    x = X_ref[0, 0]
    y = _conv_plane(x, w1_ref, b1_ref, 5)
    h = x + jax.nn.relu(y.reshape(128, 128))
    y = jax.nn.relu(_conv_plane(h, w2_ref, b2_ref, 5))
    h = _pool(y.reshape(8, 128, 128), p1)
    h = _conv_cmaj(h, w3_ref, b3_ref, 3, True)
    h = _conv_cmaj(h, w4_ref, b4_ref, 3, False)
    h = _pool(h, p2)
    h = _conv_cmaj(h, w5_ref, b5_ref, 3, True)
    h = _conv_cmaj(h, w6_ref, b6_ref, 3, False)
    enc_ref[0] = _pool(h, p3)
    flat = enc_ref[0].reshape(32, 256).T
    x2 = jnp.sum(flat * flat, axis=-1, keepdims=True)
    NCODE = emb_ref.shape[0]
    CHUNK = 2048

    def step(i, carry):
        best, besti = carry
        c0 = i * CHUNK
        embc = emb_ref[pl.ds(c0, CHUNK), :]
        m = jnp.dot(flat, embT_ref[:, pl.ds(c0, CHUNK)],
                    preferred_element_type=_F32, precision=_HI)
        e2 = jnp.sum(embc * embc, axis=-1)
        d2 = x2 - 2.0 * m + e2[None, :]
        cm = jnp.min(d2, axis=-1)
        ca = jnp.argmin(d2, axis=-1).astype(jnp.int32) + c0
        take = cm < best
        return jnp.where(take, cm, best), jnp.where(take, ca, besti)

    init = (jnp.full((256,), jnp.inf, _F32), jnp.zeros((256,), jnp.int32))
    _, besti = lax.fori_loop(0, NCODE // CHUNK, step, init)
    idx_ref[0, 0] = besti


def _encoder_vq(X, ops, emb, embT):
    B = X.shape[0]
    full = lambda a: pl.BlockSpec(a.shape, lambda i: (0,) * a.ndim)
    in_specs = [pl.BlockSpec((1, 1, 128, 128), lambda i: (i, 0, 0, 0))]
    in_specs += [full(a) for a in ops] + [full(emb), full(embT)]
    return pl.pallas_call(
        _enc_body,
        grid=(B,),
        in_specs=in_specs,
        out_specs=[pl.BlockSpec((1, 32, 16, 16), lambda i: (i, 0, 0, 0)),
                   pl.BlockSpec((1, 1, 256), lambda i: (i, 0, 0))],
        out_shape=[jax.ShapeDtypeStruct((B, 32, 16, 16), _F32),
                   jax.ShapeDtypeStruct((B, 1, 256), jnp.int32)],
        scratch_shapes=[pltpu.VMEM((8, 128, 64), _F32),
                        pltpu.VMEM((16, 64, 32), _F32),
                        pltpu.VMEM((32, 32, 16), _F32)],
    )(X, *ops, emb, embT)


def _sc_gather(emb, idx_flat):
    B = idx_flat.shape[0]
    D = emb.shape[1]
    info = plsc.get_sparse_core_info()
    NW = info.num_cores * info.num_subcores
    b_per_w = B // NW
    mesh = plsc.VectorSubcoreMesh(core_axis_name="c", subcore_axis_name="s")

    CH = 128
    n_chunks = b_per_w // CH

    @functools.partial(
        pl.kernel, mesh=mesh,
        out_type=jax.ShapeDtypeStruct((B, D), _F32),
        scratch_types=[
            pltpu.VMEM((n_chunks, CH), jnp.int32),
            pltpu.VMEM((b_per_w, D), _F32),
            pltpu.SemaphoreType.DMA,
        ],
    )
    def k(table_hbm, idx_hbm, out_hbm, idx_v, rows_v, sem):
        wid = lax.axis_index("s") * info.num_cores + lax.axis_index("c")
        base = wid * b_per_w
        for j in range(n_chunks):
            pltpu.sync_copy(idx_hbm.at[pl.ds(base + j * CH, CH)], idx_v.at[j])
        copies = [
            pltpu.async_copy(table_hbm.at[idx_v.at[j]],
                             rows_v.at[pl.ds(j * CH, CH)], sem)
            for j in range(n_chunks)]
        for c in copies:
            c.wait()
        pltpu.sync_copy(rows_v, out_hbm.at[pl.ds(base, b_per_w)])

    return k(emb, idx_flat)


def _conv_nmaj(x, H, W, wT_ref, b_ref, K, accum):
    N, C = x.shape
    p = K // 2
    Wp = W + 2 * p
    xf = jnp.pad(x.reshape(H, W, C),
                 ((p, p + 1), (p, p), (0, 0))).reshape((H + 2 * p + 1) * Wp, C)
    NN = H * Wp
    if accum:
        y = None
        for t, (ky, kx) in enumerate(_taps(K)):
            sl = xf[ky * Wp + kx: ky * Wp + kx + NN, :]
            part = jnp.dot(sl, wT_ref[t * C:(t + 1) * C, :],
                           preferred_element_type=_F32, precision=_HI)
            y = part if y is None else y + part
    else:
        pats = jnp.concatenate(
            [xf[ky * Wp + kx: ky * Wp + kx + NN, :] for ky, kx in _taps(K)],
            axis=1)
        y = jnp.dot(pats, wT_ref[...], preferred_element_type=_F32, precision=_HI)
    y = y + b_ref[...]
    Cout = y.shape[1]
    y = jax.nn.relu(y.reshape(H, Wp, Cout)[:, :W, :].reshape(H * W, Cout))
    return x + y


def _tconv_nmaj(x, H, W, t_ref, b_ref, r_ref, o_ref):
    for di in (0, 1):
        r_ref[0::2, :] = jnp.dot(x, t_ref[di, 0], preferred_element_type=_F32, precision=_HI)
        r_ref[1::2, :] = jnp.dot(x, t_ref[di, 1], preferred_element_type=_F32, precision=_HI)
        row = r_ref[...].reshape(H, 2 * W, r_ref.shape[1])
        o_ref[di::2, :, :] = jax.nn.relu(row + b_ref[...][None])
    o = o_ref[...]
    return o.reshape(4 * H * W, o.shape[2])


def _dec_body(enc_ref, din_ref, w1_ref, b1_ref, t1_ref, tb1_ref,
              w2_ref, b2_ref, t2_ref, tb2_ref, w3_ref, b3_ref,
              t3_ref, tb3_ref, dec_ref, r1, o1, r2, o2, s3):
    enc = enc_ref[0]
    din = din_ref[0]
    ste = enc + (din - enc)
    h = ste.reshape(32, 256).T
    h = _conv_nmaj(h, 16, 16, w1_ref, b1_ref, 3, False)
    h = _tconv_nmaj(h, 16, 16, t1_ref, tb1_ref, r1, o1)
    h = _conv_nmaj(h, 32, 32, w2_ref, b2_ref, 3, False)
    h = _tconv_nmaj(h, 32, 32, t2_ref, tb2_ref, r2, o2)
    h = _conv_nmaj(h, 64, 64, w3_ref, b3_ref, 5, True)
    xT3 = h.T.reshape(8, 64, 64)
    r = lax.broadcasted_iota(jnp.int32, (64, 128), 0)
    c = lax.broadcasted_iota(jnp.int32, (64, 128), 1)
    E0 = (c == 2 * r).astype(_F32)
    E1 = (c == 2 * r + 1).astype(_F32)
    for di in (0, 1):
        A = sum(t3_ref[di, 0, ch] * xT3[ch] for ch in range(8))
        Bv = sum(t3_ref[di, 1, ch] * xT3[ch] for ch in range(8))
        R = (jnp.dot(A, E0, preferred_element_type=_F32,
                     precision=lax.Precision.HIGHEST)
             + jnp.dot(Bv, E1, preferred_element_type=_F32,
                       precision=lax.Precision.HIGHEST))
        s3[di::2, :] = jax.nn.relu(R + tb3_ref[...])
    dec_ref[0, 0] = s3[...]


def _decoder(enc, dec_in, ops):
    B = enc.shape[0]
    full = lambda a: pl.BlockSpec(a.shape, lambda i: (0,) * a.ndim)
    in_specs = [pl.BlockSpec((1, 32, 16, 16), lambda i: (i, 0, 0, 0)),
                pl.BlockSpec((1, 32, 16, 16), lambda i: (i, 0, 0, 0))]
    in_specs += [full(a) for a in ops]
    return pl.pallas_call(
        _dec_body,
        grid=(B,),
        in_specs=in_specs,
        out_specs=pl.BlockSpec((1, 1, 128, 128), lambda i: (i, 0, 0, 0)),
        out_shape=jax.ShapeDtypeStruct((B, 1, 128, 128), _F32),
        scratch_shapes=[pltpu.VMEM((512, 16), _F32),
                        pltpu.VMEM((32, 32, 16), _F32),
                        pltpu.VMEM((2048, 8), _F32),
                        pltpu.VMEM((64, 64, 8), _F32),
                        pltpu.VMEM((128, 128), _F32)],
    )(enc, dec_in, *ops)


def _prep_conv(w):
    Cout, Cin, K, _ = w.shape
    return w.transpose(0, 2, 3, 1).reshape(Cout, K * K * Cin)


def _prep_tconv_n(w):
    return w.transpose(2, 3, 1, 0)[::-1, ::-1]


def kernel(X, e_res1_w, e_res1_b, e_conv1_w, e_conv1_b, e_res2_w, e_res2_b,
           e_conv2_w, e_conv2_b, e_res3_w, e_res3_b, e_conv3_w, e_conv3_b,
           emb, d_res1_w, d_res1_b, d_tconv1_w, d_tconv1_b, d_res2_w,
           d_res2_b, d_tconv2_w, d_tconv2_b, d_res3_w, d_res3_b, d_tconv3_w,
           d_tconv3_b):
    B = X.shape[0]
    eops = []
    for w, b in zip((e_res1_w, e_conv1_w, e_res2_w, e_conv2_w, e_res3_w,
                     e_conv3_w),
                    (e_res1_b, e_conv1_b, e_res2_b, e_conv2_b, e_res3_b,
                     e_conv3_b)):
        eops += [_prep_conv(w), b.reshape(-1, 1)]
    enc, idx = _encoder_vq(X, eops, emb, emb.T)
    emb_pad = jnp.pad(emb, ((0, 0), (0, 128 - emb.shape[1])))
    rows = _sc_gather(emb_pad, idx.reshape(B * 256))
    dec_in = rows[:, :32].reshape(B, 16, 16, 32).transpose(0, 3, 1, 2)
    dops = [_prep_conv(d_res1_w).T, d_res1_b.reshape(1, -1),
            _prep_tconv_n(d_tconv1_w), d_tconv1_b.reshape(1, -1),
            _prep_conv(d_res2_w).T, d_res2_b.reshape(1, -1),
            _prep_tconv_n(d_tconv2_w), d_tconv2_b.reshape(1, -1),
            _prep_conv(d_res3_w).T, d_res3_b.reshape(1, -1),
            d_tconv3_w.transpose(2, 3, 0, 1)[::-1, ::-1].reshape(2, 2, 8),
            d_tconv3_b.reshape(1, 1)]
    dec = _decoder(enc, dec_in, dops)
    return (enc, dec_in, dec)


# speedup vs baseline: 1.2503x; 1.2503x over previous
import functools

import jax
import jax.numpy as jnp
from jax import lax
from jax.experimental import pallas as pl
from jax.experimental.pallas import tpu as pltpu
from jax.experimental.pallas import tpu_sc as plsc

_F32 = jnp.float32
_HI = lax.Precision.DEFAULT


def _taps(K):
    return [(ky, kx) for ky in range(K) for kx in range(K)]


def _conv_plane(x, w_ref, b_ref, K):
    H, W = x.shape
    p = K // 2
    xp = jnp.pad(x, ((p, p), (p, p)))
    pats = jnp.stack([xp[ky:ky + H, kx:kx + W] for ky, kx in _taps(K)],
                     axis=0).reshape(K * K, H * W)
    return jnp.dot(w_ref[...], pats, preferred_element_type=_F32, precision=_HI) + b_ref[...]


def _conv_cmaj(x, w_ref, b_ref, K, res):
    C, H, W = x.shape
    p = K // 2
    Wp = W + 2 * p
    xf = jnp.pad(x, ((0, 0), (p, p + 1), (p, p))).reshape(C, (H + 2 * p + 1) * Wp)
    NN = H * Wp
    pats = jnp.concatenate(
        [xf[:, ky * Wp + kx: ky * Wp + kx + NN] for ky, kx in _taps(K)], axis=0)
    y = jnp.dot(w_ref[...], pats, preferred_element_type=_F32, precision=_HI) + b_ref[...]
    y = jax.nn.relu(y.reshape(y.shape[0], H, Wp)[:, :, :W])
    return x + y if res else y


def _pool(x, s_ref):
    C, H, W = x.shape
    mh = jnp.maximum(x[:, :, :W - 1], x[:, :, 1:])
    r = lax.broadcasted_iota(jnp.int32, (W - 1, W // 2), 0)
    c = lax.broadcasted_iota(jnp.int32, (W - 1, W // 2), 1)
    S = (r == 2 * c).astype(_F32)
    s_ref[...] = jnp.dot(mh.reshape(C * H, W - 1), S,
                         preferred_element_type=_F32,
                         precision=lax.Precision.HIGHEST).reshape(C, H, W // 2)
    return jnp.maximum(s_ref[:, ::2, :], s_ref[:, 1::2, :])


def _enc_body(X_ref, w1_ref, b1_ref, w2_ref, b2_ref, w3_ref, b3_ref,
              w4_ref, b4_ref, w5_ref, b5_ref, w6_ref, b6_ref,
              emb_ref, embT_ref, enc_ref, idx_ref, p1, p2, p3):
    x = X_ref[0, 0]
    y = _conv_plane(x, w1_ref, b1_ref, 5)
    h = x + jax.nn.relu(y.reshape(128, 128))
    y = jax.nn.relu(_conv_plane(h, w2_ref, b2_ref, 5))
    h = _pool(y.reshape(8, 128, 128), p1)
    h = _conv_cmaj(h, w3_ref, b3_ref, 3, True)
    h = _conv_cmaj(h, w4_ref, b4_ref, 3, False)
    h = _pool(h, p2)
    h = _conv_cmaj(h, w5_ref, b5_ref, 3, True)
    h = _conv_cmaj(h, w6_ref, b6_ref, 3, False)
    enc_ref[0] = _pool(h, p3)
    flat = enc_ref[0].reshape(32, 256).T
    x2 = jnp.sum(flat * flat, axis=-1, keepdims=True)
    NCODE = emb_ref.shape[0]
    CHUNK = 2048

    def step(i, carry):
        best, besti = carry
        c0 = i * CHUNK
        embc = emb_ref[pl.ds(c0, CHUNK), :]
        m = jnp.dot(flat, embT_ref[:, pl.ds(c0, CHUNK)],
                    preferred_element_type=_F32, precision=_HI)
        e2 = jnp.sum(embc * embc, axis=-1)
        d2 = x2 - 2.0 * m + e2[None, :]
        cm = jnp.min(d2, axis=-1)
        ca = jnp.argmin(d2, axis=-1).astype(jnp.int32) + c0
        take = cm < best
        return jnp.where(take, cm, best), jnp.where(take, ca, besti)

    init = (jnp.full((256,), jnp.inf, _F32), jnp.zeros((256,), jnp.int32))
    _, besti = lax.fori_loop(0, NCODE // CHUNK, step, init)
    idx_ref[0, 0] = besti


def _encoder_vq(X, ops, emb, embT):
    B = X.shape[0]
    full = lambda a: pl.BlockSpec(a.shape, lambda i: (0,) * a.ndim)
    in_specs = [pl.BlockSpec((1, 1, 128, 128), lambda i: (i, 0, 0, 0))]
    in_specs += [full(a) for a in ops] + [full(emb), full(embT)]
    return pl.pallas_call(
        _enc_body,
        grid=(B,),
        in_specs=in_specs,
        out_specs=[pl.BlockSpec((1, 32, 16, 16), lambda i: (i, 0, 0, 0)),
                   pl.BlockSpec((1, 1, 256), lambda i: (i, 0, 0))],
        out_shape=[jax.ShapeDtypeStruct((B, 32, 16, 16), _F32),
                   jax.ShapeDtypeStruct((B, 1, 256), jnp.int32)],
        scratch_shapes=[pltpu.VMEM((8, 128, 64), _F32),
                        pltpu.VMEM((16, 64, 32), _F32),
                        pltpu.VMEM((32, 32, 16), _F32)],
    )(X, *ops, emb, embT)


def _sc_gather(emb, idx_flat):
    B = idx_flat.shape[0]
    D = emb.shape[1]
    info = plsc.get_sparse_core_info()
    NW = info.num_cores * info.num_subcores
    b_per_w = B // NW
    mesh = plsc.VectorSubcoreMesh(core_axis_name="c", subcore_axis_name="s")

    CH = 128
    n_chunks = b_per_w // CH

    @functools.partial(
        pl.kernel, mesh=mesh,
        compiler_params=pltpu.CompilerParams(use_tc_tiling_on_sc=False),
        out_type=jax.ShapeDtypeStruct((B, D), _F32),
        scratch_types=[
            pltpu.VMEM((n_chunks, CH), jnp.int32),
            pltpu.VMEM((b_per_w, D), _F32),
            pltpu.SemaphoreType.DMA,
        ],
    )
    def k(table_hbm, idx_hbm, out_hbm, idx_v, rows_v, sem):
        wid = lax.axis_index("s") * info.num_cores + lax.axis_index("c")
        base = wid * b_per_w
        for j in range(n_chunks):
            pltpu.sync_copy(idx_hbm.at[pl.ds(base + j * CH, CH)], idx_v.at[j])
        copies = [
            pltpu.async_copy(table_hbm.at[idx_v.at[j]],
                             rows_v.at[pl.ds(j * CH, CH)], sem)
            for j in range(n_chunks)]
        for c in copies:
            c.wait()
        pltpu.sync_copy(rows_v, out_hbm.at[pl.ds(base, b_per_w)])

    return k(emb, idx_flat)


def _conv_nmaj(x, H, W, wT_ref, b_ref, K, accum):
    N, C = x.shape
    p = K // 2
    Wp = W + 2 * p
    xf = jnp.pad(x.reshape(H, W, C),
                 ((p, p + 1), (p, p), (0, 0))).reshape((H + 2 * p + 1) * Wp, C)
    NN = H * Wp
    if accum:
        y = None
        for t, (ky, kx) in enumerate(_taps(K)):
            sl = xf[ky * Wp + kx: ky * Wp + kx + NN, :]
            part = jnp.dot(sl, wT_ref[t * C:(t + 1) * C, :],
                           preferred_element_type=_F32, precision=_HI)
            y = part if y is None else y + part
    else:
        pats = jnp.concatenate(
            [xf[ky * Wp + kx: ky * Wp + kx + NN, :] for ky, kx in _taps(K)],
            axis=1)
        y = jnp.dot(pats, wT_ref[...], preferred_element_type=_F32, precision=_HI)
    y = y + b_ref[...]
    Cout = y.shape[1]
    y = jax.nn.relu(y.reshape(H, Wp, Cout)[:, :W, :].reshape(H * W, Cout))
    return x + y


def _tconv_nmaj(x, H, W, t_ref, b_ref, r_ref, o_ref):
    for di in (0, 1):
        r_ref[0::2, :] = jnp.dot(x, t_ref[di, 0], preferred_element_type=_F32, precision=_HI)
        r_ref[1::2, :] = jnp.dot(x, t_ref[di, 1], preferred_element_type=_F32, precision=_HI)
        row = r_ref[...].reshape(H, 2 * W, r_ref.shape[1])
        o_ref[di::2, :, :] = jax.nn.relu(row + b_ref[...][None])
    o = o_ref[...]
    return o.reshape(4 * H * W, o.shape[2])


def _dec_body(enc_ref, din_ref, w1_ref, b1_ref, t1_ref, tb1_ref,
              w2_ref, b2_ref, t2_ref, tb2_ref, w3_ref, b3_ref,
              t3_ref, tb3_ref, dec_ref, r1, o1, r2, o2, s3):
    enc = enc_ref[0]
    din = din_ref[0]
    ste = enc + (din - enc)
    h = ste.reshape(32, 256).T
    h = _conv_nmaj(h, 16, 16, w1_ref, b1_ref, 3, False)
    h = _tconv_nmaj(h, 16, 16, t1_ref, tb1_ref, r1, o1)
    h = _conv_nmaj(h, 32, 32, w2_ref, b2_ref, 3, False)
    h = _tconv_nmaj(h, 32, 32, t2_ref, tb2_ref, r2, o2)
    h = _conv_nmaj(h, 64, 64, w3_ref, b3_ref, 5, True)
    xT3 = h.T.reshape(8, 64, 64)
    r = lax.broadcasted_iota(jnp.int32, (64, 128), 0)
    c = lax.broadcasted_iota(jnp.int32, (64, 128), 1)
    E0 = (c == 2 * r).astype(_F32)
    E1 = (c == 2 * r + 1).astype(_F32)
    for di in (0, 1):
        A = sum(t3_ref[di, 0, ch] * xT3[ch] for ch in range(8))
        Bv = sum(t3_ref[di, 1, ch] * xT3[ch] for ch in range(8))
        R = (jnp.dot(A, E0, preferred_element_type=_F32,
                     precision=lax.Precision.HIGHEST)
             + jnp.dot(Bv, E1, preferred_element_type=_F32,
                       precision=lax.Precision.HIGHEST))
        s3[di::2, :] = jax.nn.relu(R + tb3_ref[...])
    dec_ref[0, 0] = s3[...]


def _decoder(enc, dec_in, ops):
    B = enc.shape[0]
    full = lambda a: pl.BlockSpec(a.shape, lambda i: (0,) * a.ndim)
    in_specs = [pl.BlockSpec((1, 32, 16, 16), lambda i: (i, 0, 0, 0)),
                pl.BlockSpec((1, 32, 16, 16), lambda i: (i, 0, 0, 0))]
    in_specs += [full(a) for a in ops]
    return pl.pallas_call(
        _dec_body,
        grid=(B,),
        in_specs=in_specs,
        out_specs=pl.BlockSpec((1, 1, 128, 128), lambda i: (i, 0, 0, 0)),
        out_shape=jax.ShapeDtypeStruct((B, 1, 128, 128), _F32),
        scratch_shapes=[pltpu.VMEM((512, 16), _F32),
                        pltpu.VMEM((32, 32, 16), _F32),
                        pltpu.VMEM((2048, 8), _F32),
                        pltpu.VMEM((64, 64, 8), _F32),
                        pltpu.VMEM((128, 128), _F32)],
    )(enc, dec_in, *ops)


def _prep_conv(w):
    Cout, Cin, K, _ = w.shape
    return w.transpose(0, 2, 3, 1).reshape(Cout, K * K * Cin)


def _prep_tconv_n(w):
    return w.transpose(2, 3, 1, 0)[::-1, ::-1]


def kernel(X, e_res1_w, e_res1_b, e_conv1_w, e_conv1_b, e_res2_w, e_res2_b,
           e_conv2_w, e_conv2_b, e_res3_w, e_res3_b, e_conv3_w, e_conv3_b,
           emb, d_res1_w, d_res1_b, d_tconv1_w, d_tconv1_b, d_res2_w,
           d_res2_b, d_tconv2_w, d_tconv2_b, d_res3_w, d_res3_b, d_tconv3_w,
           d_tconv3_b):
    B = X.shape[0]
    eops = []
    for w, b in zip((e_res1_w, e_conv1_w, e_res2_w, e_conv2_w, e_res3_w,
                     e_conv3_w),
                    (e_res1_b, e_conv1_b, e_res2_b, e_conv2_b, e_res3_b,
                     e_conv3_b)):
        eops += [_prep_conv(w), b.reshape(-1, 1)]
    enc, idx = _encoder_vq(X, eops, emb, emb.T)
    rows = _sc_gather(emb, idx.reshape(B * 256))
    dec_in = rows.reshape(B, 16, 16, 32).transpose(0, 3, 1, 2)
    dops = [_prep_conv(d_res1_w).T, d_res1_b.reshape(1, -1),
            _prep_tconv_n(d_tconv1_w), d_tconv1_b.reshape(1, -1),
            _prep_conv(d_res2_w).T, d_res2_b.reshape(1, -1),
            _prep_tconv_n(d_tconv2_w), d_tconv2_b.reshape(1, -1),
            _prep_conv(d_res3_w).T, d_res3_b.reshape(1, -1),
            d_tconv3_w.transpose(2, 3, 0, 1)[::-1, ::-1].reshape(2, 2, 8),
            d_tconv3_b.reshape(1, 1)]
    dec = _decoder(enc, dec_in, dops)
    return (enc, dec_in, dec)


# speedup vs baseline: 2.0083x; 1.6063x over previous
import functools

import jax
import jax.numpy as jnp
from jax import lax
from jax.experimental import pallas as pl
from jax.experimental.pallas import tpu as pltpu
from jax.experimental.pallas import tpu_sc as plsc

_F32 = jnp.float32
_NBE = 2
_EXACT = lax.Precision.HIGHEST
_NBD = 4
_HI = lax.Precision.DEFAULT


def _taps(K):
    return [(ky, kx) for ky in range(K) for kx in range(K)]


def _conv_plane(x, w_ref, b_ref, K):
    H, W = x.shape
    p = K // 2
    xp = jnp.pad(x, ((p, p), (p, p)))
    pats = jnp.stack([xp[ky:ky + H, kx:kx + W] for ky, kx in _taps(K)],
                     axis=0).reshape(K * K, H * W)
    return jnp.dot(w_ref[...], pats, preferred_element_type=_F32, precision=_HI) + b_ref[...]


def _conv_cmaj(x, w_ref, b_ref, K, res, nb):
    C, NH, W = x.shape
    H = NH // nb
    p = K // 2
    Wp = W + 2 * p
    Hp = H + 2 * p + 1
    xf = jnp.pad(x.reshape(C, nb, H, W),
                 ((0, 0), (0, 0), (p, p + 1), (p, p))).reshape(C, nb * Hp * Wp)
    NN = H * Wp
    pats = jnp.concatenate(
        [jnp.concatenate(
            [xf[:, n * Hp * Wp + ky * Wp + kx:
                n * Hp * Wp + ky * Wp + kx + NN] for n in range(nb)], axis=1)
         for ky, kx in _taps(K)], axis=0)
    y = jnp.dot(w_ref[...], pats, preferred_element_type=_F32) + b_ref[...]
    y = jax.nn.relu(y.reshape(y.shape[0], nb * H, Wp)[:, :, :W])
    return x + y if res else y


def _pool(x, s_ref):
    C, H, W = x.shape
    mh = jnp.maximum(x[:, :, :W - 1], x[:, :, 1:])
    r = lax.broadcasted_iota(jnp.int32, (W - 1, W // 2), 0)
    c = lax.broadcasted_iota(jnp.int32, (W - 1, W // 2), 1)
    S = (r == 2 * c).astype(_F32)
    s_ref[...] = jnp.dot(mh.reshape(C * H, W - 1), S,
                         preferred_element_type=_F32,
                         precision=lax.Precision.HIGHEST).reshape(C, H, W // 2)
    return jnp.maximum(s_ref[:, ::2, :], s_ref[:, 1::2, :])


def _enc_body(X_ref, w1_ref, b1_ref, w2_ref, b2_ref, w3_ref, b3_ref,
              w4_ref, b4_ref, w5_ref, b5_ref, w6_ref, b6_ref,
              emb_ref, embT_ref, enc_ref, idx_ref, s0, p1, p2, p3):
    nb = _NBE
    for n in range(nb):
        x = X_ref[n, 0]
        y = _conv_plane(x, w1_ref, b1_ref, 5)
        h = x + jax.nn.relu(y.reshape(128, 128))
        y = jax.nn.relu(_conv_plane(h, w2_ref, b2_ref, 5))
        s0[:, n * 128:(n + 1) * 128, :] = y.reshape(8, 128, 128)
    h = _pool(s0[...], p1)
    h = _conv_cmaj(h, w3_ref, b3_ref, 3, True, nb)
    h = _conv_cmaj(h, w4_ref, b4_ref, 3, False, nb)
    h = _pool(h, p2)
    h = _conv_cmaj(h, w5_ref, b5_ref, 3, True, nb)
    h = _conv_cmaj(h, w6_ref, b6_ref, 3, False, nb)
    enc = _pool(h, p3)
    for n in range(nb):
        enc_ref[n] = enc[:, n * 16:(n + 1) * 16, :]
    flat = jnp.concatenate(
        [enc[:, n * 16:(n + 1) * 16, :].reshape(32, 256) for n in range(nb)],
        axis=1).T
    x2 = jnp.sum(flat * flat, axis=-1, keepdims=True)
    NCODE = emb_ref.shape[0]
    CHUNK = 2048

    def step(i, carry):
        best, besti = carry
        c0 = i * CHUNK
        embc = emb_ref[pl.ds(c0, CHUNK), :]
        m = jnp.dot(flat, embT_ref[:, pl.ds(c0, CHUNK)],
                    preferred_element_type=_F32)
        e2 = jnp.sum(embc * embc, axis=-1)
        d2 = x2 - 2.0 * m + e2[None, :]
        cm = jnp.min(d2, axis=-1)
        ca = jnp.argmin(d2, axis=-1).astype(jnp.int32) + c0
        take = cm < best
        return jnp.where(take, cm, best), jnp.where(take, ca, besti)

    init = (jnp.full((nb * 256,), jnp.inf, _F32),
            jnp.zeros((nb * 256,), jnp.int32))
    _, besti = lax.fori_loop(0, NCODE // CHUNK, step, init)
    for n in range(nb):
        idx_ref[n, 0] = besti[n * 256:(n + 1) * 256]


def _encoder_vq(X, ops, emb, embT):
    B = X.shape[0]
    nb = _NBE
    full = lambda a: pl.BlockSpec(a.shape, lambda i: (0,) * a.ndim)
    in_specs = [pl.BlockSpec((nb, 1, 128, 128), lambda i: (i, 0, 0, 0))]
    in_specs += [full(a) for a in ops] + [full(emb), full(embT)]
    return pl.pallas_call(
        _enc_body,
        grid=(B // nb,),
        in_specs=in_specs,
        out_specs=[pl.BlockSpec((nb, 32, 16, 16), lambda i: (i, 0, 0, 0)),
                   pl.BlockSpec((nb, 1, 256), lambda i: (i, 0, 0))],
        out_shape=[jax.ShapeDtypeStruct((B, 32, 16, 16), _F32),
                   jax.ShapeDtypeStruct((B, 1, 256), jnp.int32)],
        scratch_shapes=[pltpu.VMEM((8, nb * 128, 128), _F32),
                        pltpu.VMEM((8, nb * 128, 64), _F32),
                        pltpu.VMEM((16, nb * 64, 32), _F32),
                        pltpu.VMEM((32, nb * 32, 16), _F32)],
    )(X, *ops, emb, embT)


def _sc_gather(emb, idx_flat):
    B = idx_flat.shape[0]
    D = emb.shape[1]
    info = plsc.get_sparse_core_info()
    NW = info.num_cores * info.num_subcores
    b_per_w = B // NW
    mesh = plsc.VectorSubcoreMesh(core_axis_name="c", subcore_axis_name="s")

    CH = 128
    n_chunks = b_per_w // CH

    @functools.partial(
        pl.kernel, mesh=mesh,
        compiler_params=pltpu.CompilerParams(use_tc_tiling_on_sc=False),
        out_type=jax.ShapeDtypeStruct((B, D), _F32),
        scratch_types=[
            pltpu.VMEM((n_chunks, CH), jnp.int32),
            pltpu.VMEM((b_per_w, D), _F32),
            pltpu.SemaphoreType.DMA,
        ],
    )
    def k(table_hbm, idx_hbm, out_hbm, idx_v, rows_v, sem):
        wid = lax.axis_index("s") * info.num_cores + lax.axis_index("c")
        base = wid * b_per_w
        for j in range(n_chunks):
            pltpu.sync_copy(idx_hbm.at[pl.ds(base + j * CH, CH)], idx_v.at[j])
        copies = [
            pltpu.async_copy(table_hbm.at[idx_v.at[j]],
                             rows_v.at[pl.ds(j * CH, CH)], sem)
            for j in range(n_chunks)]
        for c in copies:
            c.wait()
        pltpu.sync_copy(rows_v, out_hbm.at[pl.ds(base, b_per_w)])

    return k(emb, idx_flat)


def _conv_nmaj(x, H, W, wT_ref, b_ref, K, accum):
    N, C = x.shape
    p = K // 2
    Wp = W + 2 * p
    xf = jnp.pad(x.reshape(H, W, C),
                 ((p, p + 1), (p, p), (0, 0))).reshape((H + 2 * p + 1) * Wp, C)
    NN = H * Wp
    if accum:
        y = None
        for t, (ky, kx) in enumerate(_taps(K)):
            sl = xf[ky * Wp + kx: ky * Wp + kx + NN, :]
            part = jnp.dot(sl, wT_ref[t * C:(t + 1) * C, :],
                           preferred_element_type=_F32, precision=_HI)
            y = part if y is None else y + part
    else:
        pats = jnp.concatenate(
            [xf[ky * Wp + kx: ky * Wp + kx + NN, :] for ky, kx in _taps(K)],
            axis=1)
        y = jnp.dot(pats, wT_ref[...], preferred_element_type=_F32, precision=_HI)
    y = y + b_ref[...]
    Cout = y.shape[1]
    y = jax.nn.relu(y.reshape(H, Wp, Cout)[:, :W, :].reshape(H * W, Cout))
    return x + y


def _tconv_nmaj(x, H, W, t_ref, b_ref, r_ref, o_ref):
    for di in (0, 1):
        r_ref[0::2, :] = jnp.dot(x, t_ref[di, 0], preferred_element_type=_F32, precision=_HI)
        r_ref[1::2, :] = jnp.dot(x, t_ref[di, 1], preferred_element_type=_F32, precision=_HI)
        row = r_ref[...].reshape(H, 2 * W, r_ref.shape[1])
        o_ref[di::2, :, :] = jax.nn.relu(row + b_ref[...][None])
    o = o_ref[...]
    return o.reshape(4 * H * W, o.shape[2])


def _dec_body(enc_ref, din_ref, w1_ref, b1_ref, t1_ref, tb1_ref,
              w2_ref, b2_ref, t2_ref, tb2_ref, w3_ref, b3_ref,
              t3_ref, tb3_ref, dec_ref, r1, o1, r2, o2, s3):
    nb = _NBD
    enc = enc_ref[...]
    din = din_ref[...]
    ste = enc + (din - enc)
    h = jnp.concatenate([ste[n].reshape(32, 256).T for n in range(nb)],
                        axis=1)
    h = _conv_nmaj(h, 16, 16, w1_ref, b1_ref, 3, False)
    h = _tconv_nmaj(h, 16, 16, t1_ref, tb1_ref, r1, o1)
    h = _conv_nmaj(h, 32, 32, w2_ref, b2_ref, 3, False)
    h = _tconv_nmaj(h, 32, 32, t2_ref, tb2_ref, r2, o2)
    h = _conv_nmaj(h, 64, 64, w3_ref, b3_ref, 5, True)
    xT3 = h.T.reshape(nb * 8, 64, 64)
    r = lax.broadcasted_iota(jnp.int32, (64, 128), 0)
    c = lax.broadcasted_iota(jnp.int32, (64, 128), 1)
    E0 = (c == 2 * r).astype(_F32)
    E1 = (c == 2 * r + 1).astype(_F32)
    for n in range(nb):
        for di in (0, 1):
            A = sum(t3_ref[di, 0, ch] * xT3[n * 8 + ch] for ch in range(8))
            Bv = sum(t3_ref[di, 1, ch] * xT3[n * 8 + ch] for ch in range(8))
            R = (jnp.dot(A, E0, preferred_element_type=_F32, precision=_EXACT)
                 + jnp.dot(Bv, E1, preferred_element_type=_F32,
                           precision=_EXACT))
            s3[n, di::2, :] = jax.nn.relu(R + tb3_ref[...])
        dec_ref[n, 0] = s3[n]


def _decoder(enc, dec_in, ops):
    B = enc.shape[0]
    nb = _NBD
    full = lambda a: pl.BlockSpec(a.shape, lambda i: (0,) * a.ndim)
    in_specs = [pl.BlockSpec((nb, 32, 16, 16), lambda i: (i, 0, 0, 0)),
                pl.BlockSpec((nb, 32, 16, 16), lambda i: (i, 0, 0, 0))]
    in_specs += [full(a) for a in ops]
    return pl.pallas_call(
        _dec_body,
        grid=(B // nb,),
        in_specs=in_specs,
        out_specs=pl.BlockSpec((nb, 1, 128, 128), lambda i: (i, 0, 0, 0)),
        out_shape=jax.ShapeDtypeStruct((B, 1, 128, 128), _F32),
        scratch_shapes=[pltpu.VMEM((512, nb * 16), _F32),
                        pltpu.VMEM((32, 32, nb * 16), _F32),
                        pltpu.VMEM((2048, nb * 8), _F32),
                        pltpu.VMEM((64, 64, nb * 8), _F32),
                        pltpu.VMEM((nb, 128, 128), _F32)],
    )(enc, dec_in, *ops)


def _prep_conv(w):
    Cout, Cin, K, _ = w.shape
    return w.transpose(0, 2, 3, 1).reshape(Cout, K * K * Cin)


def _prep_tconv_n(w):
    return w.transpose(2, 3, 1, 0)[::-1, ::-1]


def _pack_conv(w, nb):
    Cout, Cin, K, _ = w.shape
    wt = w.transpose(2, 3, 1, 0).reshape(K * K, Cin, Cout)
    eye = jnp.eye(nb, dtype=w.dtype)
    big = wt[:, None, :, None, :] * eye[None, :, None, :, None]
    return big.reshape(K * K * nb * Cin, nb * Cout)


def _pack_tconv(w, nb):
    t = w.transpose(2, 3, 1, 0)[::-1, ::-1]
    Cin, Cout = t.shape[2], t.shape[3]
    eye = jnp.eye(nb, dtype=w.dtype)
    big = t[:, :, None, :, None, :] * eye[None, None, :, None, :, None]
    return big.reshape(2, 2, nb * Cin, nb * Cout)


def kernel(X, e_res1_w, e_res1_b, e_conv1_w, e_conv1_b, e_res2_w, e_res2_b,
           e_conv2_w, e_conv2_b, e_res3_w, e_res3_b, e_conv3_w, e_conv3_b,
           emb, d_res1_w, d_res1_b, d_tconv1_w, d_tconv1_b, d_res2_w,
           d_res2_b, d_tconv2_w, d_tconv2_b, d_res3_w, d_res3_b, d_tconv3_w,
           d_tconv3_b):
    B = X.shape[0]
    eops = []
    for w, b in zip((e_res1_w, e_conv1_w, e_res2_w, e_conv2_w, e_res3_w,
                     e_conv3_w),
                    (e_res1_b, e_conv1_b, e_res2_b, e_conv2_b, e_res3_b,
                     e_conv3_b)):
        eops += [_prep_conv(w), b.reshape(-1, 1)]
    enc, idx = _encoder_vq(X, eops, emb, emb.T)
    rows = _sc_gather(emb, idx.reshape(B * 256))
    dec_in = rows.reshape(B, 16, 16, 32).transpose(0, 3, 1, 2)
    nbd = _NBD
    dops = [_pack_conv(d_res1_w, nbd), jnp.tile(d_res1_b.reshape(1, -1), (1, nbd)),
            _pack_tconv(d_tconv1_w, nbd), jnp.tile(d_tconv1_b.reshape(1, -1), (1, nbd)),
            _pack_conv(d_res2_w, nbd), jnp.tile(d_res2_b.reshape(1, -1), (1, nbd)),
            _pack_tconv(d_tconv2_w, nbd), jnp.tile(d_tconv2_b.reshape(1, -1), (1, nbd)),
            _pack_conv(d_res3_w, nbd), jnp.tile(d_res3_b.reshape(1, -1), (1, nbd)),
            d_tconv3_w.transpose(2, 3, 0, 1)[::-1, ::-1].reshape(2, 2, 8),
            d_tconv3_b.reshape(1, 1)]
    dec = _decoder(enc, dec_in, dops)
    return (enc, dec_in, dec)


# speedup vs baseline: 2.1008x; 1.0460x over previous
import functools

import jax
import jax.numpy as jnp
from jax import lax
from jax.experimental import pallas as pl
from jax.experimental.pallas import tpu as pltpu
from jax.experimental.pallas import tpu_sc as plsc

_F32 = jnp.float32
_NBE = 2
_EXACT = lax.Precision.HIGHEST
_NBD = 4
_HI = lax.Precision.DEFAULT


def _taps(K):
    return [(ky, kx) for ky in range(K) for kx in range(K)]


def _conv_plane(x, w_ref, b_ref, K):
    H, W = x.shape
    p = K // 2
    xp = jnp.pad(x, ((p, p), (p, p)))
    pats = jnp.stack([xp[ky:ky + H, kx:kx + W] for ky, kx in _taps(K)],
                     axis=0).reshape(K * K, H * W)
    return jnp.dot(w_ref[...], pats, preferred_element_type=_F32, precision=_HI) + b_ref[...]


def _conv_cmaj(x, w_ref, b_ref, K, res, nb):
    C, NH, W = x.shape
    H = NH // nb
    p = K // 2
    Wp = W + 2 * p
    Hp = H + 2 * p + 1
    xf = jnp.pad(x.reshape(C, nb, H, W),
                 ((0, 0), (0, 0), (p, p + 1), (p, p))).reshape(C, nb * Hp * Wp)
    NN = H * Wp
    pats = jnp.concatenate(
        [jnp.concatenate(
            [xf[:, n * Hp * Wp + ky * Wp + kx:
                n * Hp * Wp + ky * Wp + kx + NN] for n in range(nb)], axis=1)
         for ky, kx in _taps(K)], axis=0)
    y = jnp.dot(w_ref[...], pats, preferred_element_type=_F32) + b_ref[...]
    y = jax.nn.relu(y.reshape(y.shape[0], nb * H, Wp)[:, :, :W])
    return x + y if res else y


def _pool(x, s_ref):
    C, H, W = x.shape
    mh = jnp.maximum(x[:, :, :W - 1], x[:, :, 1:])
    r = lax.broadcasted_iota(jnp.int32, (W - 1, W // 2), 0)
    c = lax.broadcasted_iota(jnp.int32, (W - 1, W // 2), 1)
    S = (r == 2 * c).astype(_F32)
    s_ref[...] = jnp.dot(mh.reshape(C * H, W - 1), S,
                         preferred_element_type=_F32,
                         precision=lax.Precision.HIGHEST).reshape(C, H, W // 2)
    return jnp.maximum(s_ref[:, ::2, :], s_ref[:, 1::2, :])


def _enc_body(X_ref, w1_ref, b1_ref, w2_ref, b2_ref, w3_ref, b3_ref,
              w4_ref, b4_ref, w5_ref, b5_ref, w6_ref, b6_ref,
              emb_ref, embT_ref, enc_ref, idx_ref, s0, p1, p2, p3):
    nb = _NBE
    for n in range(nb):
        x = X_ref[n, 0]
        y = _conv_plane(x, w1_ref, b1_ref, 5)
        h = x + jax.nn.relu(y.reshape(128, 128))
        y = jax.nn.relu(_conv_plane(h, w2_ref, b2_ref, 5))
        s0[:, n * 128:(n + 1) * 128, :] = y.reshape(8, 128, 128)
    h = _pool(s0[...], p1)
    h = _conv_cmaj(h, w3_ref, b3_ref, 3, True, nb)
    h = _conv_cmaj(h, w4_ref, b4_ref, 3, False, nb)
    h = _pool(h, p2)
    h = _conv_cmaj(h, w5_ref, b5_ref, 3, True, nb)
    h = _conv_cmaj(h, w6_ref, b6_ref, 3, False, nb)
    enc = _pool(h, p3)
    for n in range(nb):
        enc_ref[n] = enc[:, n * 16:(n + 1) * 16, :]
    flat = jnp.concatenate(
        [enc[:, n * 16:(n + 1) * 16, :].reshape(32, 256) for n in range(nb)],
        axis=1).T
    x2 = jnp.sum(flat * flat, axis=-1, keepdims=True)
    NCODE = emb_ref.shape[0]
    CHUNK = 2048

    def step(i, carry):
        best, besti = carry
        c0 = i * CHUNK
        embc = emb_ref[pl.ds(c0, CHUNK), :]
        m = jnp.dot(flat, embT_ref[:, pl.ds(c0, CHUNK)],
                    preferred_element_type=_F32)
        e2 = jnp.sum(embc * embc, axis=-1)
        d2 = x2 - 2.0 * m + e2[None, :]
        cm = jnp.min(d2, axis=-1)
        ca = jnp.argmin(d2, axis=-1).astype(jnp.int32) + c0
        take = cm < best
        return jnp.where(take, cm, best), jnp.where(take, ca, besti)

    init = (jnp.full((nb * 256,), jnp.inf, _F32),
            jnp.zeros((nb * 256,), jnp.int32))
    _, besti = lax.fori_loop(0, NCODE // CHUNK, step, init)
    for n in range(nb):
        idx_ref[n, 0] = besti[n * 256:(n + 1) * 256]


def _encoder_vq(X, ops, emb, embT):
    B = X.shape[0]
    nb = _NBE
    full = lambda a: pl.BlockSpec(a.shape, lambda i: (0,) * a.ndim)
    in_specs = [pl.BlockSpec((nb, 1, 128, 128), lambda i: (i, 0, 0, 0))]
    in_specs += [full(a) for a in ops] + [full(emb), full(embT)]
    return pl.pallas_call(
        _enc_body,
        grid=(B // nb,),
        in_specs=in_specs,
        out_specs=[pl.BlockSpec((nb, 32, 16, 16), lambda i: (i, 0, 0, 0)),
                   pl.BlockSpec((nb, 1, 256), lambda i: (i, 0, 0))],
        out_shape=[jax.ShapeDtypeStruct((B, 32, 16, 16), _F32),
                   jax.ShapeDtypeStruct((B, 1, 256), jnp.int32)],
        scratch_shapes=[pltpu.VMEM((8, nb * 128, 128), _F32),
                        pltpu.VMEM((8, nb * 128, 64), _F32),
                        pltpu.VMEM((16, nb * 64, 32), _F32),
                        pltpu.VMEM((32, nb * 32, 16), _F32)],
    )(X, *ops, emb, embT)


def _sc_gather(emb, idx_flat):
    B = idx_flat.shape[0]
    D = emb.shape[1]
    info = plsc.get_sparse_core_info()
    NW = info.num_cores * info.num_subcores
    b_per_w = B // NW
    mesh = plsc.VectorSubcoreMesh(core_axis_name="c", subcore_axis_name="s")

    CH = 128
    n_chunks = b_per_w // CH

    @functools.partial(
        pl.kernel, mesh=mesh,
        compiler_params=pltpu.CompilerParams(use_tc_tiling_on_sc=False),
        out_type=jax.ShapeDtypeStruct((B, D), _F32),
        scratch_types=[
            pltpu.VMEM((n_chunks, CH), jnp.int32),
            pltpu.VMEM((b_per_w, D), _F32),
            pltpu.SemaphoreType.DMA,
        ],
    )
    def k(table_hbm, idx_hbm, out_hbm, idx_v, rows_v, sem):
        wid = lax.axis_index("s") * info.num_cores + lax.axis_index("c")
        base = wid * b_per_w
        for j in range(n_chunks):
            pltpu.sync_copy(idx_hbm.at[pl.ds(base + j * CH, CH)], idx_v.at[j])
        copies = [
            pltpu.async_copy(table_hbm.at[idx_v.at[j]],
                             rows_v.at[pl.ds(j * CH, CH)], sem)
            for j in range(n_chunks)]
        for c in copies:
            c.wait()
        pltpu.sync_copy(rows_v, out_hbm.at[pl.ds(base, b_per_w)])

    return k(emb, idx_flat)


def _conv_nmaj(x, H, W, wT_ref, b_ref, K, accum):
    N, C = x.shape
    p = K // 2
    Wp = W + 2 * p
    xf = jnp.pad(x.reshape(H, W, C),
                 ((p, p + 1), (p, p), (0, 0))).reshape((H + 2 * p + 1) * Wp, C)
    NN = H * Wp
    if accum:
        y = None
        for t, (ky, kx) in enumerate(_taps(K)):
            sl = xf[ky * Wp + kx: ky * Wp + kx + NN, :]
            part = jnp.dot(sl, wT_ref[t * C:(t + 1) * C, :],
                           preferred_element_type=_F32, precision=_HI)
            y = part if y is None else y + part
    else:
        pats = jnp.concatenate(
            [xf[ky * Wp + kx: ky * Wp + kx + NN, :] for ky, kx in _taps(K)],
            axis=1)
        y = jnp.dot(pats, wT_ref[...], preferred_element_type=_F32, precision=_HI)
    y = y + b_ref[...]
    Cout = y.shape[1]
    y = jax.nn.relu(y.reshape(H, Wp, Cout)[:, :W, :].reshape(H * W, Cout))
    return x + y


def _tconv_nmaj(x, H, W, t_ref, b_ref, r_ref, o_ref):
    for di in (0, 1):
        r_ref[0::2, :] = jnp.dot(x, t_ref[di, 0], preferred_element_type=_F32, precision=_HI)
        r_ref[1::2, :] = jnp.dot(x, t_ref[di, 1], preferred_element_type=_F32, precision=_HI)
        row = r_ref[...].reshape(H, 2 * W, r_ref.shape[1])
        o_ref[di::2, :, :] = jax.nn.relu(row + b_ref[...][None])
    o = o_ref[...]
    return o.reshape(4 * H * W, o.shape[2])


def _dec_body(enc_ref, din_ref, w1_ref, b1_ref, t1_ref, tb1_ref,
              w2_ref, b2_ref, t2_ref, tb2_ref, w3_ref, b3_ref,
              t3_ref, tb3_ref, dec_ref, r1, o1, r2, o2, s3):
    nb = _NBD
    enc = enc_ref[...]
    din = din_ref[...]
    ste = enc + (din - enc)
    h = jnp.concatenate([ste[n].reshape(32, 256).T for n in range(nb)],
                        axis=1)
    h = _conv_nmaj(h, 16, 16, w1_ref, b1_ref, 3, False)
    h = _tconv_nmaj(h, 16, 16, t1_ref, tb1_ref, r1, o1)
    h = _conv_nmaj(h, 32, 32, w2_ref, b2_ref, 3, False)
    h = _tconv_nmaj(h, 32, 32, t2_ref, tb2_ref, r2, o2)
    h = _conv_nmaj(h, 64, 64, w3_ref, b3_ref, 5, True)
    xT3 = h.T.reshape(nb * 8, 64, 64)
    r = lax.broadcasted_iota(jnp.int32, (64, 128), 0)
    c = lax.broadcasted_iota(jnp.int32, (64, 128), 1)
    E0 = (c == 2 * r).astype(_F32)
    E1 = (c == 2 * r + 1).astype(_F32)
    for n in range(nb):
        for di in (0, 1):
            A = sum(t3_ref[di, 0, ch] * xT3[n * 8 + ch] for ch in range(8))
            Bv = sum(t3_ref[di, 1, ch] * xT3[n * 8 + ch] for ch in range(8))
            R = (jnp.dot(A, E0, preferred_element_type=_F32, precision=_EXACT)
                 + jnp.dot(Bv, E1, preferred_element_type=_F32,
                           precision=_EXACT))
            s3[n, di::2, :] = jax.nn.relu(R + tb3_ref[...])
        dec_ref[n, 0] = s3[n]


def _decoder(enc, dec_in, ops):
    B = enc.shape[0]
    nb = _NBD
    full = lambda a: pl.BlockSpec(a.shape, lambda i: (0,) * a.ndim)
    in_specs = [pl.BlockSpec((nb, 32, 16, 16), lambda i: (i, 0, 0, 0)),
                pl.BlockSpec((nb, 32, 16, 16), lambda i: (i, 0, 0, 0))]
    in_specs += [full(a) for a in ops]
    return pl.pallas_call(
        _dec_body,
        grid=(B // nb,),
        in_specs=in_specs,
        out_specs=pl.BlockSpec((nb, 1, 128, 128), lambda i: (i, 0, 0, 0)),
        out_shape=jax.ShapeDtypeStruct((B, 1, 128, 128), _F32),
        scratch_shapes=[pltpu.VMEM((512, nb * 16), _F32),
                        pltpu.VMEM((32, 32, nb * 16), _F32),
                        pltpu.VMEM((2048, nb * 8), _F32),
                        pltpu.VMEM((64, 64, nb * 8), _F32),
                        pltpu.VMEM((nb, 128, 128), _F32)],
    )(enc, dec_in, *ops)


def _prep_conv(w):
    Cout, Cin, K, _ = w.shape
    return w.transpose(0, 2, 3, 1).reshape(Cout, K * K * Cin)


def _prep_tconv_n(w):
    return w.transpose(2, 3, 1, 0)[::-1, ::-1]


def _pack_conv(w, nb):
    Cout, Cin, K, _ = w.shape
    wt = w.transpose(2, 3, 1, 0).reshape(K * K, Cin, Cout)
    eye = jnp.eye(nb, dtype=w.dtype)
    big = wt[:, None, :, None, :] * eye[None, :, None, :, None]
    return big.reshape(K * K * nb * Cin, nb * Cout)


def _pack_tconv(w, nb):
    t = w.transpose(2, 3, 1, 0)[::-1, ::-1]
    Cin, Cout = t.shape[2], t.shape[3]
    eye = jnp.eye(nb, dtype=w.dtype)
    big = t[:, :, None, :, None, :] * eye[None, None, :, None, :, None]
    return big.reshape(2, 2, nb * Cin, nb * Cout)


def kernel(X, e_res1_w, e_res1_b, e_conv1_w, e_conv1_b, e_res2_w, e_res2_b,
           e_conv2_w, e_conv2_b, e_res3_w, e_res3_b, e_conv3_w, e_conv3_b,
           emb, d_res1_w, d_res1_b, d_tconv1_w, d_tconv1_b, d_res2_w,
           d_res2_b, d_tconv2_w, d_tconv2_b, d_res3_w, d_res3_b, d_tconv3_w,
           d_tconv3_b):
    B = X.shape[0]
    eops = []
    for w, b in zip((e_res1_w, e_conv1_w, e_res2_w, e_conv2_w, e_res3_w,
                     e_conv3_w),
                    (e_res1_b, e_conv1_b, e_res2_b, e_conv2_b, e_res3_b,
                     e_conv3_b)):
        eops += [_prep_conv(w), b.reshape(-1, 1)]
    embT = emb.T
    H = B // 2
    enc1, idx1 = _encoder_vq(X[:H], eops, emb, embT)
    rows1 = _sc_gather(emb, idx1.reshape(H * 256))
    enc2, idx2 = _encoder_vq(X[H:], eops, emb, embT)
    rows2 = _sc_gather(emb, idx2.reshape(H * 256))
    din1 = rows1.reshape(H, 16, 16, 32).transpose(0, 3, 1, 2)
    din2 = rows2.reshape(H, 16, 16, 32).transpose(0, 3, 1, 2)
    nbd = _NBD
    dops = [_pack_conv(d_res1_w, nbd), jnp.tile(d_res1_b.reshape(1, -1), (1, nbd)),
            _pack_tconv(d_tconv1_w, nbd), jnp.tile(d_tconv1_b.reshape(1, -1), (1, nbd)),
            _pack_conv(d_res2_w, nbd), jnp.tile(d_res2_b.reshape(1, -1), (1, nbd)),
            _pack_tconv(d_tconv2_w, nbd), jnp.tile(d_tconv2_b.reshape(1, -1), (1, nbd)),
            _pack_conv(d_res3_w, nbd), jnp.tile(d_res3_b.reshape(1, -1), (1, nbd)),
            d_tconv3_w.transpose(2, 3, 0, 1)[::-1, ::-1].reshape(2, 2, 8),
            d_tconv3_b.reshape(1, 1)]
    dec1 = _decoder(enc1, din1, dops)
    dec2 = _decoder(enc2, din2, dops)
    enc = jnp.concatenate([enc1, enc2], axis=0)
    dec_in = jnp.concatenate([din1, din2], axis=0)
    dec = jnp.concatenate([dec1, dec2], axis=0)
    return (enc, dec_in, dec)
